# Initial kernel scaffold; baseline (speedup 1.0000x reference)
import jax
import jax.numpy as jnp
from jax.experimental import pallas as pl


def kernel(seg_map, class_embeddings):
    raise NotImplementedError("write your pallas kernel here")


# speedup vs baseline: 28.9043x; 28.9043x over previous
import jax
import jax.numpy as jnp
from jax.experimental import pallas as pl

TEXT_DIM = 512
NUM_CLASSES = 20
KPAD = 32
OUT_H = 128
OUT_W = 256
IN_W = 512
ROWS_PER_STEP = 8


def _emb_kernel(seg_ref, et_ref, sel_ref, out_ref):
    et = et_ref[...]
    sel = sel_ref[...]
    kio = jax.lax.broadcasted_iota(jnp.int32, (KPAD, IN_W), 0)
    for r in range(ROWS_PER_STEP):
        row = seg_ref[0, r, 0:1, :]
        oh = (kio == row).astype(jnp.float32)
        oh_ds = jax.lax.dot_general(
            oh, sel, (((1,), (0,)), ((), ())),
            preferred_element_type=jnp.float32)
        res = jax.lax.dot_general(
            et, oh_ds, (((1,), (0,)), ((), ())),
            preferred_element_type=jnp.float32)
        out_ref[0, :, r, :] = res


@jax.jit
def kernel(seg_map, class_embeddings):
    bs = seg_map.shape[0]
    seg_r = seg_map.reshape(bs, OUT_H, 2, IN_W)
    et = jnp.zeros((TEXT_DIM, KPAD), jnp.float32)
    et = et.at[:, :NUM_CLASSES].set(class_embeddings.T)
    sel = (jax.lax.broadcasted_iota(jnp.int32, (IN_W, OUT_W), 0)
           == 2 * jax.lax.broadcasted_iota(jnp.int32, (IN_W, OUT_W), 1)
           ).astype(jnp.float32)
    grid = (bs, OUT_H // ROWS_PER_STEP)
    return pl.pallas_call(
        _emb_kernel,
        grid=grid,
        in_specs=[
            pl.BlockSpec((1, ROWS_PER_STEP, 2, IN_W), lambda b, i: (b, i, 0, 0)),
            pl.BlockSpec((TEXT_DIM, KPAD), lambda b, i: (0, 0)),
            pl.BlockSpec((IN_W, OUT_W), lambda b, i: (0, 0)),
        ],
        out_specs=pl.BlockSpec(
            (1, TEXT_DIM, ROWS_PER_STEP, OUT_W), lambda b, i: (b, 0, i, 0)),
        out_shape=jax.ShapeDtypeStruct((bs, TEXT_DIM, OUT_H, OUT_W), jnp.float32),
    )(seg_r, et, sel)
